# Initial kernel scaffold; baseline (speedup 1.0000x reference)
import jax
import jax.numpy as jnp
from jax.experimental import pallas as pl


def kernel(text, weight):
    raise NotImplementedError("write your pallas kernel here")


# speedup vs baseline: 2.0556x; 2.0556x over previous
import functools

import jax
import jax.numpy as jnp
from jax import lax
from jax.experimental import pallas as pl
from jax.experimental.pallas import tpu as pltpu
from jax.experimental.pallas import tpu_sc as plsc

NC = 2
NS = 16
NW = NC * NS
LANES = 16

CHUNK_BAGS = 2


def _make_kernel(B, H, D):
    bags_per_w = B // NW
    idx_per_chunk = CHUNK_BAGS * H
    nchunk = bags_per_w // CHUNK_BAGS
    col_groups = D // LANES
    inv_h = 1.0 / H

    mesh = plsc.VectorSubcoreMesh(core_axis_name="c", subcore_axis_name="s")

    @functools.partial(
        pl.kernel,
        out_type=jax.ShapeDtypeStruct((B, D), jnp.float32),
        mesh=mesh,
        scratch_types=[
            pltpu.VMEM((nchunk, idx_per_chunk), jnp.int32),
            pltpu.VMEM((idx_per_chunk, D), jnp.float32),
            pltpu.VMEM((bags_per_w, D), jnp.float32),
            pltpu.SemaphoreType.DMA,
        ],
        compiler_params=pltpu.CompilerParams(use_tc_tiling_on_sc=False),
    )
    def bag_kernel(text_hbm, weight_hbm, out_hbm, idx_v, rows_v, out_v, sem):
        wid = lax.axis_index("s") * NC + lax.axis_index("c")
        pltpu.sync_copy(text_hbm.at[wid], idx_v)

        def chunk_body(j, _):
            pltpu.async_copy(weight_hbm.at[idx_v.at[j]], rows_v, sem).wait()
            for bag in range(CHUNK_BAGS):
                for c in range(col_groups):
                    acc = rows_v[bag * H, pl.ds(c * LANES, LANES)]
                    for r in range(1, H):
                        acc = acc + rows_v[bag * H + r, pl.ds(c * LANES, LANES)]
                    out_v[j * CHUNK_BAGS + bag, pl.ds(c * LANES, LANES)] = acc * inv_h
            return 0

        lax.fori_loop(0, nchunk, chunk_body, 0)
        pltpu.sync_copy(out_v, out_hbm.at[pl.ds(wid * bags_per_w, bags_per_w)])

    return bag_kernel


def kernel(text, weight):
    B, H = text.shape
    _, D = weight.shape
    text_r = text.astype(jnp.int32).reshape(NW, (B // NW) // CHUNK_BAGS, CHUNK_BAGS * H)
    return _make_kernel(B, H, D)(text_r, weight)


# speedup vs baseline: 2.0904x; 1.0169x over previous
import functools

import jax
import jax.numpy as jnp
from jax import lax
from jax.experimental import pallas as pl
from jax.experimental.pallas import tpu as pltpu
from jax.experimental.pallas import tpu_sc as plsc

NC = 2
NS = 16
NW = NC * NS
LANES = 16

CHUNK_BAGS = 2


def _make_kernel(B, H, D):
    bags_per_w = B // NW
    idx_per_chunk = CHUNK_BAGS * H
    nchunk = bags_per_w // CHUNK_BAGS
    col_groups = D // LANES
    inv_h = 1.0 / H

    mesh = plsc.VectorSubcoreMesh(core_axis_name="c", subcore_axis_name="s")

    nbuf = 4
    assert nchunk % nbuf == 0

    @functools.partial(
        pl.kernel,
        out_type=jax.ShapeDtypeStruct((B, D), jnp.float32),
        mesh=mesh,
        scratch_types=[
            pltpu.VMEM((nchunk, idx_per_chunk), jnp.int32),
            pltpu.VMEM((nbuf, idx_per_chunk, D), jnp.float32),
            pltpu.VMEM((bags_per_w, D), jnp.float32),
            [pltpu.SemaphoreType.DMA] * nbuf,
        ],
        compiler_params=pltpu.CompilerParams(use_tc_tiling_on_sc=False),
    )
    def bag_kernel(text_hbm, weight_hbm, out_hbm, idx_v, rows_v, out_v, sems):
        wid = lax.axis_index("s") * NC + lax.axis_index("c")
        pltpu.sync_copy(text_hbm.at[wid], idx_v)

        def start(j, b):
            pltpu.async_copy(weight_hbm.at[idx_v.at[j]], rows_v.at[b], sems[b])

        for b in range(nbuf):
            start(b, b)

        def outer(g, _):
            j0 = g * nbuf
            for b in range(nbuf):
                j = j0 + b
                pltpu.make_async_copy(
                    weight_hbm.at[idx_v.at[j]], rows_v.at[b], sems[b]
                ).wait()
                for bag in range(CHUNK_BAGS):
                    for c in range(col_groups):
                        acc = rows_v[b, bag * H, pl.ds(c * LANES, LANES)]
                        for r in range(1, H):
                            acc = acc + rows_v[b, bag * H + r, pl.ds(c * LANES, LANES)]
                        out_v[j * CHUNK_BAGS + bag, pl.ds(c * LANES, LANES)] = acc * inv_h
                nxt = j + nbuf

                @pl.when(nxt < nchunk)
                def _():
                    start(nxt, b)

            return 0

        lax.fori_loop(0, nchunk // nbuf, outer, 0)
        pltpu.sync_copy(out_v, out_hbm.at[pl.ds(wid * bags_per_w, bags_per_w)])

    return bag_kernel


def kernel(text, weight):
    B, H = text.shape
    _, D = weight.shape
    text_r = text.astype(jnp.int32).reshape(NW, (B // NW) // CHUNK_BAGS, CHUNK_BAGS * H)
    return _make_kernel(B, H, D)(text_r, weight)
